# Initial kernel scaffold; baseline (speedup 1.0000x reference)
import jax
import jax.numpy as jnp
from jax.experimental import pallas as pl


def kernel(pred, target, x, pos, edge_attr, edge_index):
    raise NotImplementedError("write your pallas kernel here")


# speedup vs baseline: 95.5883x; 95.5883x over previous
import functools

import jax
import jax.numpy as jnp
from jax import lax
from jax.experimental import pallas as pl
from jax.experimental.pallas import tpu as pltpu
from jax.experimental.pallas import tpu_sc as plsc

RHO_L = 1000.0
RHO_G = 1.0
MU = 0.001
RHO = 1000.0
CP = 4186.0
DT = 1e-05
W_MASS = 0.1
W_MOM = 0.05
W_EN = 0.01
EPS = 1e-8

NC = 2
NS = 16
NW = NC * NS
L = 16
CB = 128
KJ = 4
C = CB * KJ

f32 = jnp.float32
i32 = jnp.int32


def _col(cc):
    return jnp.full((L,), cc, i32)


def _zero_buf(contrib):
    w = contrib.shape[1]
    lanes = lax.iota(i32, L)
    rowoff = lanes // w
    coloff = lanes % w
    rows_per = L // w
    zero = jnp.zeros((L,), f32)

    def _zb(i, _):
        plsc.store_scatter(contrib, [i * rows_per + rowoff, coloff], zero)
        return 0
    lax.fori_loop(0, contrib.shape[0] // rows_per, _zb, 0)


def _pass1_body(nch, nodeA, src_h, dst_h, out_h,
                accum, srcv, dstv, rowsS, rowsD, contrib, sem1, sem2):
    c = lax.axis_index("c")
    s = lax.axis_index("s")
    wid = c * NS + s
    nacc = out_h.shape[1]
    rps = nacc // NS
    lanes = lax.iota(i32, L)
    one = jnp.ones((L,), f32)

    _zero_buf(contrib)
    for k in range(rps // C):
        pltpu.sync_copy(contrib, accum.at[pl.ds(s * rps + k * C, C)])
    plsc.subcore_barrier()

    cols = [_col(j) for j in range(11)]

    def _chunk(ci, _):
        base = (wid * nch + ci) * KJ
        pltpu.sync_copy(src_h.at[pl.ds(base, KJ)], srcv)
        pltpu.sync_copy(dst_h.at[pl.ds(base, KJ)], dstv)
        cps = [pltpu.async_copy(nodeA.at[srcv.at[j]],
                                rowsS.at[pl.ds(j * CB, CB)], sem1)
               for j in range(KJ)]
        cpd = [pltpu.async_copy(nodeA.at[dstv.at[j]],
                                rowsD.at[pl.ds(j * CB, CB)], sem2)
               for j in range(KJ)]
        for cp in cps:
            cp.wait()
        for cp in cpd:
            cp.wait()

        def _grp(g, _):
            ridx = g * L + lanes

            def ld(ref, cc):
                return plsc.load_gather(ref, [ridx, cols[cc]])

            psx = ld(rowsS, 0); psy = ld(rowsS, 1); psz = ld(rowsS, 2)
            vsx = ld(rowsS, 3); vsy = ld(rowsS, 4); vsz = ld(rowsS, 5)
            rs = ld(rowsS, 6)
            pdx = ld(rowsD, 0); pdy = ld(rowsD, 1); pdz = ld(rowsD, 2)
            vdx = ld(rowsD, 3); vdy = ld(rowsD, 4); vdz = ld(rowsD, 5)
            rd = ld(rowsD, 6)
            dx = pdx - psx; dy = pdy - psy; dz = pdz - psz
            inv = 1.0 / (dx * dx + dy * dy + dz * dz + EPS)
            mx = rd * vdx - rs * vsx
            my = rd * vdy - rs * vsy
            mz = rd * vdz - rs * vsz
            divc = (mx * dx + my * dy + mz * dz) * inv
            wx = dx * inv; wy = dy * inv; wz = dz * inv
            ux = vdx - vsx; uy = vdy - vsy; uz = vdz - vsz

            def put(cc, val):
                plsc.store_scatter(contrib, [ridx, cols[cc]], val)

            put(0, divc)
            put(1, ux * wx); put(2, ux * wy); put(3, ux * wz)
            put(4, uy * wx); put(5, uy * wy); put(6, uy * wz)
            put(7, uz * wx); put(8, uz * wy); put(9, uz * wz)
            put(10, one)
            return 0
        lax.fori_loop(0, C // L, _grp, 0)
        for j in range(KJ):
            pltpu.sync_copy(contrib.at[pl.ds(j * CB, CB)],
                            accum.at[dstv.at[j]], add=True)
        return 0
    lax.fori_loop(0, nch, _chunk, 0)

    plsc.subcore_barrier()
    for k in range(rps // C):
        rows = pl.ds(s * rps + k * C, C)
        pltpu.sync_copy(accum.at[rows], contrib)
        pltpu.sync_copy(contrib, out_h.at[c, rows])


def _pass2_body(nch, nodeB, src_h, dst_h, out_h,
                accum, srcv, dstv, rowsS, rowsD, contrib, sem1, sem2):
    c = lax.axis_index("c")
    s = lax.axis_index("s")
    wid = c * NS + s
    nacc = out_h.shape[1]
    rps = nacc // NS
    lanes = lax.iota(i32, L)
    _zero_buf(contrib)
    for k in range(rps // C):
        pltpu.sync_copy(contrib, accum.at[pl.ds(s * rps + k * C, C)])
    plsc.subcore_barrier()

    cols = [_col(j) for j in range(12)]

    def _chunk(ci, _):
        base = (wid * nch + ci) * KJ
        pltpu.sync_copy(src_h.at[pl.ds(base, KJ)], srcv)
        pltpu.sync_copy(dst_h.at[pl.ds(base, KJ)], dstv)
        cps = [pltpu.async_copy(nodeB.at[srcv.at[j]],
                                rowsS.at[pl.ds(j * CB, CB)], sem1)
               for j in range(KJ)]
        cpd = [pltpu.async_copy(nodeB.at[dstv.at[j]],
                                rowsD.at[pl.ds(j * CB, CB)], sem2)
               for j in range(KJ)]
        for cp in cps:
            cp.wait()
        for cp in cpd:
            cp.wait()

        def _grp(g, _):
            ridx = g * L + lanes

            def ld(ref, cc):
                return plsc.load_gather(ref, [ridx, cols[cc]])

            psx = ld(rowsS, 0); psy = ld(rowsS, 1); psz = ld(rowsS, 2)
            pdx = ld(rowsD, 0); pdy = ld(rowsD, 1); pdz = ld(rowsD, 2)
            dx = pdx - psx; dy = pdy - psy; dz = pdz - psz
            inv = 1.0 / (dx * dx + dy * dy + dz * dz + EPS)

            def put(cc, val):
                plsc.store_scatter(contrib, [ridx, cols[cc]], val)

            for i in range(3):
                gsx = ld(rowsS, 3 + 3 * i); gsy = ld(rowsS, 4 + 3 * i)
                gsz = ld(rowsS, 5 + 3 * i)
                gdx = ld(rowsD, 3 + 3 * i); gdy = ld(rowsD, 4 + 3 * i)
                gdz = ld(rowsD, 5 + 3 * i)
                lap = ((gdx - gsx) * dx + (gdy - gsy) * dy
                       + (gdz - gsz) * dz) * inv
                put(i, lap)
            return 0
        lax.fori_loop(0, C // L, _grp, 0)
        for j in range(KJ):
            pltpu.sync_copy(contrib.at[pl.ds(j * CB, CB)],
                            accum.at[dstv.at[j]], add=True)
        return 0
    lax.fori_loop(0, nch, _chunk, 0)

    plsc.subcore_barrier()
    for k in range(rps // C):
        rows = pl.ds(s * rps + k * C, C)
        pltpu.sync_copy(accum.at[rows], contrib)
        pltpu.sync_copy(contrib, out_h.at[c, rows])


def _merge_kernel(a0, a1, na, nb):
    a = a0[...] + a1[...]
    deg = jnp.maximum(a[:, 10:11], 1.0)
    invd = 1.0 / deg
    g = a[:, 1:10] * invd
    divm = a[:, 0:1] * invd
    pos = na[:, 0:3]
    z = jnp.zeros_like(a[:, 0:2])
    nb[...] = jnp.concatenate([pos, g, divm, deg, z], axis=1)


def _final_kernel(pred, x, nb, b0, b1, out, acc):
    i = pl.program_id(0)
    n_total = pl.num_programs(0)

    @pl.when(i == 0)
    def _():
        acc[0] = 0.0
        acc[1] = 0.0
        acc[2] = 0.0

    du = pred[:, 2:5]
    lap = (b0[:, 0:3] + b1[:, 0:3]) / nb[:, 13:14]
    phys = (MU / RHO) * DT * lap
    divm = nb[:, 12]
    vel = x[:, 5:8]
    dke = 0.5 * RHO * (jnp.sum((vel + du) ** 2, axis=1)
                       - jnp.sum(vel ** 2, axis=1))
    de = dke + RHO * CP * pred[:, 0]
    acc[0] += jnp.sum(divm ** 2)
    acc[1] += jnp.sum((du - phys) ** 2)
    acc[2] += jnp.sum(de)

    @pl.when(i == n_total - 1)
    def _():
        n = jnp.float32(x.shape[0] * n_total)
        mass = acc[0] / n
        mom = acc[1] / (3.0 * n)
        en = (acc[2] / n) ** 2
        out[0, 0] = W_MASS * mass + W_MOM * mom + W_EN * en


def kernel(pred, target, x, pos, edge_attr, edge_index):
    N = x.shape[0]
    E = edge_index.shape[1]
    vel = x[:, 5:8]
    rho = jnp.where(x[:, 4] < 1.5, RHO_L, RHO_G).astype(f32)

    rps = C * (-(-(N + 1) // (NS * C)))
    nacc = NS * rps
    nodeA = jnp.zeros((nacc, 8), f32)
    nodeA = nodeA.at[:N, 0:3].set(pos).at[:N, 3:6].set(vel).at[:N, 6].set(rho)

    nch = -(-E // (NW * C))
    epad = NW * nch * C
    pad = jnp.full((epad - E,), N, i32)
    src = jnp.concatenate([edge_index[0], pad]).reshape(epad // CB, CB)
    dst = jnp.concatenate([edge_index[1], pad]).reshape(epad // CB, CB)

    mesh = plsc.VectorSubcoreMesh(core_axis_name="c", subcore_axis_name="s",
                                  num_cores=NC, num_subcores=NS)
    sc_scratch = lambda w: [
        pltpu.VMEM_SHARED((nacc, w), f32),
        pltpu.VMEM((KJ, CB), i32),
        pltpu.VMEM((KJ, CB), i32),
        pltpu.VMEM((C, 8 if w == 16 else 16), f32),
        pltpu.VMEM((C, 8 if w == 16 else 16), f32),
        pltpu.VMEM((C, w), f32),
        pltpu.SemaphoreType.DMA,
        pltpu.SemaphoreType.DMA,
    ]

    sc_params = pltpu.CompilerParams(needs_layout_passes=False,
                                     use_tc_tiling_on_sc=False)
    pass1 = pl.kernel(
        functools.partial(_pass1_body, nch),
        out_type=jax.ShapeDtypeStruct((NC, nacc, 16), f32),
        mesh=mesh,
        scratch_types=sc_scratch(16),
        compiler_params=sc_params,
    )
    outA = pass1(nodeA, src, dst)

    bm = 2048
    nodeB = pl.pallas_call(
        _merge_kernel,
        grid=(nacc // bm,),
        in_specs=[pl.BlockSpec((bm, 16), lambda i: (i, 0)),
                  pl.BlockSpec((bm, 16), lambda i: (i, 0)),
                  pl.BlockSpec((bm, 8), lambda i: (i, 0))],
        out_specs=pl.BlockSpec((bm, 16), lambda i: (i, 0)),
        out_shape=jax.ShapeDtypeStruct((nacc, 16), f32),
    )(outA[0], outA[1], nodeA)

    pass2 = pl.kernel(
        functools.partial(_pass2_body, nch),
        out_type=jax.ShapeDtypeStruct((NC, nacc, 8), f32),
        mesh=mesh,
        scratch_types=sc_scratch(8),
        compiler_params=sc_params,
    )
    outB = pass2(nodeB, src, dst)

    bn = 2000
    total = pl.pallas_call(
        _final_kernel,
        grid=(N // bn,),
        in_specs=[pl.BlockSpec((bn, 5), lambda i: (i, 0)),
                  pl.BlockSpec((bn, 8), lambda i: (i, 0)),
                  pl.BlockSpec((bn, 16), lambda i: (i, 0)),
                  pl.BlockSpec((bn, 8), lambda i: (i, 0)),
                  pl.BlockSpec((bn, 8), lambda i: (i, 0))],
        out_specs=pl.BlockSpec(memory_space=pltpu.SMEM),
        out_shape=jax.ShapeDtypeStruct((1, 1), f32),
        scratch_shapes=[pltpu.SMEM((4,), f32)],
    )(pred, x, nodeB, outB[0], outB[1])
    return total.reshape(())


# speedup vs baseline: 99.3098x; 1.0389x over previous
import functools

import jax
import jax.numpy as jnp
from jax import lax
from jax.experimental import pallas as pl
from jax.experimental.pallas import tpu as pltpu
from jax.experimental.pallas import tpu_sc as plsc

RHO_L = 1000.0
RHO_G = 1.0
MU = 0.001
RHO = 1000.0
CP = 4186.0
DT = 1e-05
W_MASS = 0.1
W_MOM = 0.05
W_EN = 0.01
EPS = 1e-8

NC = 2
NS = 16
NW = NC * NS
L = 16
CB = 128
KJ = 4
C = CB * KJ

f32 = jnp.float32
i32 = jnp.int32


def _col(cc):
    return jnp.full((L,), cc, i32)


def _zero_buf(contrib):
    w = contrib.shape[1]
    lanes = lax.iota(i32, L)
    rowoff = lanes // w
    coloff = lanes % w
    rows_per = L // w
    zero = jnp.zeros((L,), f32)

    def _zb(i, _):
        plsc.store_scatter(contrib, [i * rows_per + rowoff, coloff], zero)
        return 0
    lax.fori_loop(0, contrib.shape[0] // rows_per, _zb, 0)


def _pass1_body(nch, kj, nodeA, src_h, dst_h, out_h,
                accum, srcv, dstv, rowsS, rowsD, contrib, sem1, sem2):
    c = lax.axis_index("c")
    s = lax.axis_index("s")
    wid = c * NS + s
    cc = kj * CB
    nacc = out_h.shape[1]
    rps = nacc // NS
    lanes = lax.iota(i32, L)
    one = jnp.ones((L,), f32)

    _zero_buf(contrib)
    for k in range(rps // cc):
        pltpu.sync_copy(contrib, accum.at[pl.ds(s * rps + k * cc, cc)])
    plsc.subcore_barrier()

    cols = [_col(j) for j in range(11)]

    def _chunk(ci, _):
        base = (wid * nch + ci) * kj
        pltpu.sync_copy(src_h.at[pl.ds(base, kj)], srcv)
        pltpu.sync_copy(dst_h.at[pl.ds(base, kj)], dstv)
        cps = [pltpu.async_copy(nodeA.at[srcv.at[j]],
                                rowsS.at[pl.ds(j * CB, CB)], sem1)
               for j in range(kj)]
        cpd = [pltpu.async_copy(nodeA.at[dstv.at[j]],
                                rowsD.at[pl.ds(j * CB, CB)], sem2)
               for j in range(kj)]
        for cp in cps:
            cp.wait()
        for cp in cpd:
            cp.wait()

        def _grp(g, _):
            ridx = g * L + lanes

            def ld(ref, cc):
                return plsc.load_gather(ref, [ridx, cols[cc]])

            psx = ld(rowsS, 0); psy = ld(rowsS, 1); psz = ld(rowsS, 2)
            vsx = ld(rowsS, 3); vsy = ld(rowsS, 4); vsz = ld(rowsS, 5)
            rs = ld(rowsS, 6)
            pdx = ld(rowsD, 0); pdy = ld(rowsD, 1); pdz = ld(rowsD, 2)
            vdx = ld(rowsD, 3); vdy = ld(rowsD, 4); vdz = ld(rowsD, 5)
            rd = ld(rowsD, 6)
            dx = pdx - psx; dy = pdy - psy; dz = pdz - psz
            inv = 1.0 / (dx * dx + dy * dy + dz * dz + EPS)
            mx = rd * vdx - rs * vsx
            my = rd * vdy - rs * vsy
            mz = rd * vdz - rs * vsz
            divc = (mx * dx + my * dy + mz * dz) * inv
            wx = dx * inv; wy = dy * inv; wz = dz * inv
            ux = vdx - vsx; uy = vdy - vsy; uz = vdz - vsz

            def put(cc, val):
                plsc.store_scatter(contrib, [ridx, cols[cc]], val)

            put(0, divc)
            put(1, ux * wx); put(2, ux * wy); put(3, ux * wz)
            put(4, uy * wx); put(5, uy * wy); put(6, uy * wz)
            put(7, uz * wx); put(8, uz * wy); put(9, uz * wz)
            put(10, one)
            return 0
        lax.fori_loop(0, cc // L, _grp, 0)
        for j in range(kj):
            pltpu.sync_copy(contrib.at[pl.ds(j * CB, CB)],
                            accum.at[dstv.at[j]], add=True)
        return 0
    lax.fori_loop(0, nch, _chunk, 0)

    plsc.subcore_barrier()
    for k in range(rps // cc):
        rows = pl.ds(s * rps + k * cc, cc)
        pltpu.sync_copy(accum.at[rows], contrib)
        pltpu.sync_copy(contrib, out_h.at[c, rows])


def _pass2_body(nch, kj, nodeB, src_h, dst_h, out_h,
                accum, srcv, dstv, rowsS, rowsD, contrib, sem1, sem2):
    c = lax.axis_index("c")
    s = lax.axis_index("s")
    wid = c * NS + s
    cc = kj * CB
    nacc = out_h.shape[1]
    rps = nacc // NS
    lanes = lax.iota(i32, L)
    _zero_buf(contrib)
    for k in range(rps // cc):
        pltpu.sync_copy(contrib, accum.at[pl.ds(s * rps + k * cc, cc)])
    plsc.subcore_barrier()

    cols = [_col(j) for j in range(12)]

    def _chunk(ci, _):
        base = (wid * nch + ci) * kj
        pltpu.sync_copy(src_h.at[pl.ds(base, kj)], srcv)
        pltpu.sync_copy(dst_h.at[pl.ds(base, kj)], dstv)
        cps = [pltpu.async_copy(nodeB.at[srcv.at[j]],
                                rowsS.at[pl.ds(j * CB, CB)], sem1)
               for j in range(kj)]
        cpd = [pltpu.async_copy(nodeB.at[dstv.at[j]],
                                rowsD.at[pl.ds(j * CB, CB)], sem2)
               for j in range(kj)]
        for cp in cps:
            cp.wait()
        for cp in cpd:
            cp.wait()

        def _grp(g, _):
            ridx = g * L + lanes

            def ld(ref, cc):
                return plsc.load_gather(ref, [ridx, cols[cc]])

            psx = ld(rowsS, 0); psy = ld(rowsS, 1); psz = ld(rowsS, 2)
            pdx = ld(rowsD, 0); pdy = ld(rowsD, 1); pdz = ld(rowsD, 2)
            dx = pdx - psx; dy = pdy - psy; dz = pdz - psz
            inv = 1.0 / (dx * dx + dy * dy + dz * dz + EPS)

            def put(cc, val):
                plsc.store_scatter(contrib, [ridx, cols[cc]], val)

            for i in range(3):
                gsx = ld(rowsS, 3 + 3 * i); gsy = ld(rowsS, 4 + 3 * i)
                gsz = ld(rowsS, 5 + 3 * i)
                gdx = ld(rowsD, 3 + 3 * i); gdy = ld(rowsD, 4 + 3 * i)
                gdz = ld(rowsD, 5 + 3 * i)
                lap = ((gdx - gsx) * dx + (gdy - gsy) * dy
                       + (gdz - gsz) * dz) * inv
                put(i, lap)
            return 0
        lax.fori_loop(0, cc // L, _grp, 0)
        for j in range(kj):
            pltpu.sync_copy(contrib.at[pl.ds(j * CB, CB)],
                            accum.at[dstv.at[j]], add=True)
        return 0
    lax.fori_loop(0, nch, _chunk, 0)

    plsc.subcore_barrier()
    for k in range(rps // cc):
        rows = pl.ds(s * rps + k * cc, cc)
        pltpu.sync_copy(accum.at[rows], contrib)
        pltpu.sync_copy(contrib, out_h.at[c, rows])


def _merge_kernel(a0, a1, na, nb):
    a = a0[...] + a1[...]
    deg = jnp.maximum(a[:, 10:11], 1.0)
    invd = 1.0 / deg
    g = a[:, 1:10] * invd
    divm = a[:, 0:1] * invd
    pos = na[:, 0:3]
    z = jnp.zeros_like(a[:, 0:2])
    nb[...] = jnp.concatenate([pos, g, divm, deg, z], axis=1)


def _final_kernel(pred, x, nb, b0, b1, out, acc):
    i = pl.program_id(0)
    n_total = pl.num_programs(0)

    @pl.when(i == 0)
    def _():
        acc[0] = 0.0
        acc[1] = 0.0
        acc[2] = 0.0

    du = pred[:, 2:5]
    lap = (b0[:, 0:3] + b1[:, 0:3]) / nb[:, 13:14]
    phys = (MU / RHO) * DT * lap
    divm = nb[:, 12]
    vel = x[:, 5:8]
    dke = 0.5 * RHO * (jnp.sum((vel + du) ** 2, axis=1)
                       - jnp.sum(vel ** 2, axis=1))
    de = dke + RHO * CP * pred[:, 0]
    acc[0] += jnp.sum(divm ** 2)
    acc[1] += jnp.sum((du - phys) ** 2)
    acc[2] += jnp.sum(de)

    @pl.when(i == n_total - 1)
    def _():
        n = jnp.float32(x.shape[0] * n_total)
        mass = acc[0] / n
        mom = acc[1] / (3.0 * n)
        en = (acc[2] / n) ** 2
        out[0, 0] = W_MASS * mass + W_MOM * mom + W_EN * en


def kernel(pred, target, x, pos, edge_attr, edge_index):
    N = x.shape[0]
    E = edge_index.shape[1]
    vel = x[:, 5:8]
    rho = jnp.where(x[:, 4] < 1.5, RHO_L, RHO_G).astype(f32)

    kj1, kj2 = 4, 8
    c1, c2 = kj1 * CB, kj2 * CB
    rps1 = c1 * (-(-(N + 1) // (NS * c1)))
    nacc1 = NS * rps1
    rps2 = c2 * (-(-(N + 1) // (NS * c2)))
    nacc2 = NS * rps2
    nodeA = jnp.zeros((nacc1, 8), f32)
    nodeA = nodeA.at[:N, 0:3].set(pos).at[:N, 3:6].set(vel).at[:N, 6].set(rho)

    nch1 = -(-E // (NW * c1))
    nch2 = -(-E // (NW * c2))
    epad = max(NW * nch1 * c1, NW * nch2 * c2)
    pad = jnp.full((epad - E,), N, i32)
    src = jnp.concatenate([edge_index[0], pad]).reshape(epad // CB, CB)
    dst = jnp.concatenate([edge_index[1], pad]).reshape(epad // CB, CB)

    mesh = plsc.VectorSubcoreMesh(core_axis_name="c", subcore_axis_name="s",
                                  num_cores=NC, num_subcores=NS)
    sc_scratch = lambda w, kj, nacc: [
        pltpu.VMEM_SHARED((nacc, w), f32),
        pltpu.VMEM((kj, CB), i32),
        pltpu.VMEM((kj, CB), i32),
        pltpu.VMEM((kj * CB, 8 if w == 16 else 16), f32),
        pltpu.VMEM((kj * CB, 8 if w == 16 else 16), f32),
        pltpu.VMEM((kj * CB, w), f32),
        pltpu.SemaphoreType.DMA,
        pltpu.SemaphoreType.DMA,
    ]

    sc_params = pltpu.CompilerParams(needs_layout_passes=False,
                                     use_tc_tiling_on_sc=False)
    pass1 = pl.kernel(
        functools.partial(_pass1_body, nch1, kj1),
        out_type=jax.ShapeDtypeStruct((NC, nacc1, 16), f32),
        mesh=mesh,
        scratch_types=sc_scratch(16, kj1, nacc1),
        compiler_params=sc_params,
    )
    outA = pass1(nodeA, src, dst)

    bm = 2048
    nodeB = pl.pallas_call(
        _merge_kernel,
        grid=(nacc1 // bm,),
        in_specs=[pl.BlockSpec((bm, 16), lambda i: (i, 0)),
                  pl.BlockSpec((bm, 16), lambda i: (i, 0)),
                  pl.BlockSpec((bm, 8), lambda i: (i, 0))],
        out_specs=pl.BlockSpec((bm, 16), lambda i: (i, 0)),
        out_shape=jax.ShapeDtypeStruct((nacc1, 16), f32),
    )(outA[0], outA[1], nodeA)

    pass2 = pl.kernel(
        functools.partial(_pass2_body, nch2, kj2),
        out_type=jax.ShapeDtypeStruct((NC, nacc2, 8), f32),
        mesh=mesh,
        scratch_types=sc_scratch(8, kj2, nacc2),
        compiler_params=sc_params,
    )
    outB = pass2(nodeB, src, dst)

    bn = 2000
    total = pl.pallas_call(
        _final_kernel,
        grid=(N // bn,),
        in_specs=[pl.BlockSpec((bn, 5), lambda i: (i, 0)),
                  pl.BlockSpec((bn, 8), lambda i: (i, 0)),
                  pl.BlockSpec((bn, 16), lambda i: (i, 0)),
                  pl.BlockSpec((bn, 8), lambda i: (i, 0)),
                  pl.BlockSpec((bn, 8), lambda i: (i, 0))],
        out_specs=pl.BlockSpec(memory_space=pltpu.SMEM),
        out_shape=jax.ShapeDtypeStruct((1, 1), f32),
        scratch_shapes=[pltpu.SMEM((4,), f32)],
    )(pred, x, nodeB, outB[0], outB[1])
    return total.reshape(())
